# baseline (speedup 1.0000x reference)
import jax
import jax.numpy as jnp
from jax import lax
from jax.experimental import pallas as pl
from jax.experimental.pallas import tpu as pltpu
from jax.experimental.pallas import tpu_sc as plsc

HEAD_SIZE = 128
HALF = 64
NUM_Q_HEADS = 32
NUM_KV_HEADS = 8

_NC, _NS = 2, 16
_NW = _NC * _NS


def _sc_gather(positions, comb_table, tokens, offset):
    rows = tokens // _NW
    ns = -(-rows // 128)
    p = rows // ns
    assert p * ns == rows and p % 8 == 0 and p <= 128

    def body(pos_hbm, tab_hbm, out_hbm, idx_v, buf, sem):
        wid = lax.axis_index("s") * _NC + lax.axis_index("c")
        for j in range(ns):
            pltpu.sync_copy(
                pos_hbm.at[pl.ds(offset + wid * rows + j * p, p)],
                idx_v.at[j])
        copies = [
            pltpu.async_copy(
                tab_hbm.at[idx_v.at[j]], buf.at[pl.ds(j * p, p)], sem)
            for j in range(ns)
        ]
        for c in copies:
            c.wait()
        pltpu.sync_copy(buf, out_hbm.at[pl.ds(wid * rows, rows)])

    mesh = plsc.VectorSubcoreMesh(core_axis_name="c", subcore_axis_name="s",
                                  num_cores=_NC, num_subcores=_NS)
    f = pl.kernel(
        body,
        out_type=jax.ShapeDtypeStruct((tokens, HEAD_SIZE), jnp.float32),
        mesh=mesh,
        scratch_types=[
            pltpu.VMEM((ns, p), jnp.int32),
            pltpu.VMEM((rows, HEAD_SIZE), jnp.float32),
            pltpu.SemaphoreType.DMA,
        ],
    )
    return f(positions, comb_table)


def _sc_key_rotary(key, cs_g):
    T, KW = key.shape
    rows = T // _NW
    SUB = 16
    G = rows // SUB
    NH = KW // HEAD_SIZE

    def body(key_hbm, cs_hbm, out_hbm, kb, csb, ob, sin, sout):
        wid = lax.axis_index("s") * _NC + lax.axis_index("c")
        base = wid * rows

        def start_in(g):
            b = g % 2
            return (pltpu.async_copy(key_hbm.at[pl.ds(base + g * SUB, SUB)],
                                     kb.at[b], sin[b]),
                    pltpu.async_copy(cs_hbm.at[pl.ds(base + g * SUB, SUB)],
                                     csb.at[b], sin[b]))

        ins = [start_in(0), None]
        outs = [None, None]
        for g in range(G):
            b = g % 2
            if g + 1 < G:
                ins[(g + 1) % 2] = start_in(g + 1)
            for cp in ins[b]:
                cp.wait()
            if outs[b] is not None:
                outs[b].wait()

            def slab(t, _):
                for v in range(HALF // 16):
                    c = csb[b, t, pl.ds(v * 16, 16)]
                    s = csb[b, t, pl.ds(HALF + v * 16, 16)]
                    for h in range(NH):
                        o = h * HEAD_SIZE + v * 16
                        x1 = kb[b, t, pl.ds(o, 16)]
                        x2 = kb[b, t, pl.ds(o + HALF, 16)]
                        ob[b, t, pl.ds(o, 16)] = x1 * c - x2 * s
                        ob[b, t, pl.ds(o + HALF, 16)] = x2 * c + x1 * s
                return _

            lax.fori_loop(0, SUB, slab, 0, unroll=False)
            outs[b] = pltpu.async_copy(
                ob.at[b], out_hbm.at[pl.ds(base + g * SUB, SUB)], sout[b])
        for cp in outs:
            if cp is not None:
                cp.wait()

    mesh = plsc.VectorSubcoreMesh(core_axis_name="c", subcore_axis_name="s",
                                  num_cores=_NC, num_subcores=_NS)
    f = pl.kernel(
        body,
        out_type=jax.ShapeDtypeStruct((T, KW), jnp.float32),
        mesh=mesh,
        scratch_types=[
            pltpu.VMEM((2, SUB, KW), jnp.float32),
            pltpu.VMEM((2, SUB, HEAD_SIZE), jnp.float32),
            pltpu.VMEM((2, SUB, KW), jnp.float32),
            [pltpu.SemaphoreType.DMA, pltpu.SemaphoreType.DMA],
            [pltpu.SemaphoreType.DMA, pltpu.SemaphoreType.DMA],
        ],
    )
    return f(key, cs_g)


def _apply_body(cs_ref, q_ref, qo_ref):
    cs = cs_ref[...]
    c = cs[:, :HALF]
    s = cs[:, HALF:]
    cc = jnp.concatenate([c, c], axis=1)
    ss = jnp.concatenate([-s, s], axis=1)
    for h in range(NUM_Q_HEADS):
        x = q_ref[:, h * HEAD_SIZE:(h + 1) * HEAD_SIZE]
        xs = jnp.concatenate([x[:, HALF:], x[:, :HALF]], axis=1)
        qo_ref[:, h * HEAD_SIZE:(h + 1) * HEAD_SIZE] = x * cc + xs * ss


def _tc_apply(cs_g, q, block_t):
    nb = q.shape[0] // block_t
    cs_spec = pl.BlockSpec((block_t, HEAD_SIZE), lambda i: (i, 0))
    q_spec = pl.BlockSpec((block_t, q.shape[1]), lambda i: (i, 0))
    return pl.pallas_call(
        _apply_body,
        grid=(nb,),
        in_specs=[cs_spec, q_spec],
        out_specs=q_spec,
        out_shape=jax.ShapeDtypeStruct(q.shape, jnp.float32),
        compiler_params=pltpu.CompilerParams(
            dimension_semantics=("arbitrary",)),
    )(cs_g, q)


@jax.jit
def kernel(positions, query, key, cos_cache, sin_cache):
    T = positions.shape[0]
    comb = jnp.concatenate([cos_cache[:, :HALF], sin_cache[:, :HALF]], axis=1)
    cs_g = _sc_gather(positions, comb, tokens=T, offset=0)
    key_out = _sc_key_rotary(key, cs_g)
    query_out = _tc_apply(cs_g, query, block_t=512)
    return query_out, key_out


# speedup vs baseline: 1.0294x; 1.0294x over previous
import jax
import jax.numpy as jnp
from jax import lax
from jax.experimental import pallas as pl
from jax.experimental.pallas import tpu as pltpu
from jax.experimental.pallas import tpu_sc as plsc

HEAD_SIZE = 128
HALF = 64
NUM_Q_HEADS = 32
NUM_KV_HEADS = 8

_NC, _NS = 2, 16
_NW = _NC * _NS


def _sc_gather(positions, comb_table):
    tokens = positions.shape[0]
    rows = tokens // _NW
    ns = -(-rows // 128)
    p = rows // ns
    assert p * ns == rows and p % 8 == 0 and p <= 128

    def body(pos_hbm, tab_hbm, out_hbm, idx_v, buf, sem):
        wid = lax.axis_index("s") * _NC + lax.axis_index("c")
        for j in range(ns):
            pltpu.sync_copy(pos_hbm.at[pl.ds(wid * rows + j * p, p)],
                            idx_v.at[j])
        copies = [
            pltpu.async_copy(
                tab_hbm.at[idx_v.at[j]], buf.at[pl.ds(j * p, p)], sem)
            for j in range(ns)
        ]
        for c in copies:
            c.wait()
        pltpu.sync_copy(buf, out_hbm.at[pl.ds(wid * rows, rows)])

    mesh = plsc.VectorSubcoreMesh(core_axis_name="c", subcore_axis_name="s",
                                  num_cores=_NC, num_subcores=_NS)
    f = pl.kernel(
        body,
        out_type=jax.ShapeDtypeStruct((tokens, HEAD_SIZE), jnp.float32),
        mesh=mesh,
        scratch_types=[
            pltpu.VMEM((ns, p), jnp.int32),
            pltpu.VMEM((rows, HEAD_SIZE), jnp.float32),
            pltpu.SemaphoreType.DMA,
        ],
    )
    return f(positions, comb_table)


def _apply_body(cs_ref, q_ref, k_ref, qo_ref, ko_ref):
    cs = cs_ref[...]
    c = cs[:, :HALF]
    s = cs[:, HALF:]
    cc = jnp.concatenate([c, c], axis=1)
    ss = jnp.concatenate([-s, s], axis=1)
    for x_ref, o_ref, heads in ((q_ref, qo_ref, NUM_Q_HEADS),
                                (k_ref, ko_ref, NUM_KV_HEADS)):
        for h in range(heads):
            x = x_ref[:, h * HEAD_SIZE:(h + 1) * HEAD_SIZE]
            xs = jnp.concatenate([x[:, HALF:], x[:, :HALF]], axis=1)
            o_ref[:, h * HEAD_SIZE:(h + 1) * HEAD_SIZE] = x * cc + xs * ss


def _tc_apply(cs_g, q, k, block_t):
    nb = q.shape[0] // block_t
    cs_spec = pl.BlockSpec((block_t, HEAD_SIZE), lambda i: (i, 0))
    q_spec = pl.BlockSpec((block_t, q.shape[1]), lambda i: (i, 0))
    k_spec = pl.BlockSpec((block_t, k.shape[1]), lambda i: (i, 0))
    return pl.pallas_call(
        _apply_body,
        grid=(nb,),
        in_specs=[cs_spec, q_spec, k_spec],
        out_specs=[q_spec, k_spec],
        out_shape=[jax.ShapeDtypeStruct(q.shape, jnp.float32),
                   jax.ShapeDtypeStruct(k.shape, jnp.float32)],
        compiler_params=pltpu.CompilerParams(
            dimension_semantics=("arbitrary",)),
    )(cs_g, q, k)


@jax.jit
def kernel(positions, query, key, cos_cache, sin_cache):
    comb = jnp.concatenate([cos_cache[:, :HALF], sin_cache[:, :HALF]], axis=1)
    cs_g = _sc_gather(positions, comb)
    return _tc_apply(cs_g, query, key, block_t=512)
